# baseline (speedup 1.0000x reference)
import functools

import jax
import jax.numpy as jnp
from jax import lax
from jax.experimental import pallas as pl
from jax.experimental.pallas import tpu as pltpu
from jax.experimental.pallas import tpu_sc as plsc

_ROWS, _COLS = 16384, 200
_D = 16
_V = 64
_G8 = _COLS // 8
_NC, _NS = 2, 16
_NW = _NC * _NS
_RPW = _ROWS // _NW
_CR = 8
_NCH = _RPW // _CR


def _make_emb():
    mesh = plsc.VectorSubcoreMesh(core_axis_name="c", subcore_axis_name="s")

    @functools.partial(
        pl.kernel,
        mesh=mesh,
        compiler_params=pltpu.CompilerParams(
            needs_layout_passes=False, disable_bounds_checks=True),
        out_type=jax.ShapeDtypeStruct((_ROWS * _G8, 128), jnp.float32),
        scratch_types=[
            pltpu.VMEM((_V, _D), jnp.float32),
            pltpu.VMEM((_CR * _COLS,), jnp.int32),
            pltpu.VMEM((_CR * _COLS,), jnp.int32),
            pltpu.VMEM((_CR * _G8, 128), jnp.float32),
            pltpu.VMEM((_CR * _G8, 128), jnp.float32),
            pltpu.SemaphoreType.DMA,
            pltpu.SemaphoreType.DMA,
            pltpu.SemaphoreType.DMA,
            pltpu.SemaphoreType.DMA,
        ],
    )
    def emb(x_hbm, table_hbm, out_hbm,
            tab_v, idx0, idx1, rows0, rows1, sin0, sin1, sout0, sout1):
        wid = lax.axis_index("s") * _NC + lax.axis_index("c")
        wbase = wid * _RPW
        pltpu.sync_copy(table_hbm, tab_v)
        idx_b = (idx0, idx1)
        rows_b = (rows0, rows1)
        sin_b = (sin0, sin1)
        sout_b = (sout0, sout1)

        def idx_src(ch):
            return x_hbm.at[pl.ds((wbase + ch * _CR) * _COLS, _CR * _COLS)]

        def out_dst(ch):
            return out_hbm.at[pl.ds((wbase + ch * _CR) * _G8, _CR * _G8), :]

        pltpu.async_copy(idx_src(0), idx0, sin0)
        pltpu.async_copy(idx_src(1), idx1, sin1)

        def chunk_pair(i, carry):
            cc = i * 2
            for b in range(2):
                ch = cc + b
                idxv, rowsv = idx_b[b], rows_b[b]
                pltpu.make_async_copy(idx_src(ch), idxv, sin_b[b]).wait()

                @pl.when(ch >= 2)
                def _():
                    pltpu.make_async_copy(rowsv, out_dst(ch - 2),
                                          sout_b[b]).wait()

                for r in range(_CR):
                    @plsc.parallel_loop(0, 13, unroll=2)
                    def _grp(j, r=r):
                        c = jnp.where(j >= 12, _COLS - 16, j * 16)
                        ivec = idxv[pl.ds(r * _COLS + c, 16)]
                        rows = [tab_v[ivec[k], :] for k in range(16)]
                        q = r * _G8 + lax.shift_right_logical(c, 3)
                        for k in range(16):
                            rowsv[q + k // 8, pl.ds((k % 8) * _D, _D)] = (
                                rows[k])

                pltpu.async_copy(rowsv, out_dst(ch), sout_b[b])

                @pl.when(ch + 2 < _NCH)
                def _():
                    pltpu.async_copy(idx_src(ch + 2), idxv, sin_b[b])
            return carry

        lax.fori_loop(0, _NCH // 2, chunk_pair, 0)
        for b in range(2):
            pltpu.make_async_copy(rows_b[b], out_dst(_NCH - 2 + b),
                                  sout_b[b]).wait()

    return emb


_emb = _make_emb()


def kernel(x, table):
    out = _emb(x.reshape(_ROWS * _COLS), table)
    return out.reshape(_ROWS, _COLS, _D)


# speedup vs baseline: 4.8230x; 4.8230x over previous
import functools

import jax
import jax.numpy as jnp
from jax import lax
from jax.experimental import pallas as pl
from jax.experimental.pallas import tpu as pltpu
from jax.experimental.pallas import tpu_sc as plsc

_ROWS, _COLS = 16384, 200
_D = 16
_W = _COLS * _D
_V = 64
_NC, _NS = 2, 16
_NW = _NC * _NS
_RPW = _ROWS // _NW
_CR = 8
_NCH = _RPW // _CR


def _make_emb():
    mesh = plsc.VectorSubcoreMesh(core_axis_name="c", subcore_axis_name="s")

    @functools.partial(
        pl.kernel,
        mesh=mesh,
        compiler_params=pltpu.CompilerParams(
            needs_layout_passes=False, disable_bounds_checks=True),
        out_type=jax.ShapeDtypeStruct((_ROWS, _W), jnp.float32),
        scratch_types=[
            pltpu.VMEM((_V, _D), jnp.float32),
            pltpu.VMEM((_CR * _COLS,), jnp.int32),
            pltpu.VMEM((_CR * _COLS,), jnp.int32),
            pltpu.VMEM((_CR, _W), jnp.float32),
            pltpu.VMEM((_CR, _W), jnp.float32),
            pltpu.SemaphoreType.DMA,
            pltpu.SemaphoreType.DMA,
            pltpu.SemaphoreType.DMA,
            pltpu.SemaphoreType.DMA,
        ],
    )
    def emb(x_hbm, table_hbm, out_hbm,
            tab_v, idx0, idx1, rows0, rows1, sin0, sin1, sout0, sout1):
        wid = lax.axis_index("s") * _NC + lax.axis_index("c")
        wbase = wid * _RPW
        pltpu.sync_copy(table_hbm, tab_v)
        idx_b = (idx0, idx1)
        rows_b = (rows0, rows1)
        sin_b = (sin0, sin1)
        sout_b = (sout0, sout1)

        def idx_src(ch):
            return x_hbm.at[pl.ds((wbase + ch * _CR) * _COLS, _CR * _COLS)]

        def out_dst(ch):
            return out_hbm.at[pl.ds(wbase + ch * _CR, _CR), :]

        pltpu.async_copy(idx_src(0), idx0, sin0)
        pltpu.async_copy(idx_src(1), idx1, sin1)

        def chunk_pair(i, carry):
            cc = i * 2
            for b in range(2):
                ch = cc + b
                idxv, rowsv = idx_b[b], rows_b[b]
                pltpu.make_async_copy(idx_src(ch), idxv, sin_b[b]).wait()

                @pl.when(ch >= 2)
                def _():
                    pltpu.make_async_copy(rowsv, out_dst(ch - 2),
                                          sout_b[b]).wait()

                for r in range(_CR):
                    @plsc.parallel_loop(0, 13, unroll=2)
                    def _grp(j, r=r):
                        c = jnp.where(j >= 12, _COLS - 16, j * 16)
                        ivec = idxv[pl.ds(r * _COLS + c, 16)]
                        rows = [tab_v[ivec[k], :] for k in range(16)]
                        for k in range(16):
                            rowsv[r, pl.ds((c + k) * _D, _D)] = rows[k]

                pltpu.async_copy(rowsv, out_dst(ch), sout_b[b])

                @pl.when(ch + 2 < _NCH)
                def _():
                    pltpu.async_copy(idx_src(ch + 2), idxv, sin_b[b])
            return carry

        lax.fori_loop(0, _NCH // 2, chunk_pair, 0)
        for b in range(2):
            pltpu.make_async_copy(rows_b[b], out_dst(_NCH - 2 + b),
                                  sout_b[b]).wait()

    return emb


_emb = _make_emb()


def kernel(x, table):
    return _emb(x.reshape(_ROWS * _COLS), table).reshape(_ROWS, _COLS, _D)
